# baseline (speedup 1.0000x reference)
import functools

import jax
import jax.numpy as jnp
from jax import lax
from jax.experimental import pallas as pl
from jax.experimental.pallas import tpu as pltpu

_K = 512
_NEG = float("-inf")


def _stage(x, d, lanes, run=None, flip=None, asc=False):
    m = x.shape[0]
    k = m // (2 * d)
    xr = x.reshape(k, 2, d, lanes)
    a = xr[:, 0]
    b = xr[:, 1]
    mx = jnp.maximum(a, b)
    mn = jnp.minimum(a, b)
    if run is None:
        lo, hi = (mn, mx) if asc else (mx, mn)
    else:
        run_shift = run.bit_length() - 1
        shift = run_shift - (2 * d).bit_length() + 1
        blk = lax.broadcasted_iota(jnp.int32, (k, 1, 1), 0)
        desc = ((blk >> shift) & 1) == 0
        if flip is not None:
            desc = desc != flip
        lo = jnp.where(desc, mx, mn)
        hi = jnp.where(desc, mn, mx)
    return jnp.concatenate([lo[:, None], hi[:, None]], axis=1).reshape(m, lanes)


def _sort_chunk(x, chunk, lanes, flip):
    run = 2
    while run <= chunk:
        d = run // 2
        while d >= 1:
            x = _stage(x, d, lanes, run=run, flip=flip)
            d //= 2
        run *= 2
    return x


def _merge_const(m, lanes, asc):
    nc, chunk, _ = m.shape
    x = m.reshape(nc * chunk, lanes)
    d = chunk // 2
    while d >= 1:
        x = _stage(x, d, lanes, asc=asc)
        d //= 2
    return x.reshape(nc, chunk, lanes)


def _merge_tree(x, chunk, nchunks, lanes):
    xr = x.reshape(nchunks // 2, 2, chunk, lanes)
    m = jnp.maximum(xr[:, 0], xr[:, 1])
    nm = nchunks // 2
    while nm > 1:
        mr = m.reshape(nm // 2, 2, chunk, lanes)
        dsc = _merge_const(mr[:, 0], lanes, asc=False)
        acs = _merge_const(mr[:, 1], lanes, asc=True)
        m = jnp.maximum(dsc, acs)
        nm //= 2
    return _merge_const(m, lanes, asc=False).reshape(chunk, lanes)


def _topk_body(len_ref, x_ref, o_ref, scratch, *, s, k, lanes):
    b = pl.program_id(0)
    length = len_ref[b]
    x = x_ref[0]
    row = lax.broadcasted_iota(jnp.int32, (s, 1), 0)
    scratch[:] = jnp.where(row < length, x, _NEG)

    nact = (length + (k - 1)) // k

    def chunk_body(c, carry):
        ch = scratch[pl.ds(c * k, k), :]
        scratch[pl.ds(c * k, k), :] = _sort_chunk(ch, k, lanes, (c & 1) == 1)
        return carry

    lax.fori_loop(0, nact, chunk_body, 0)
    y = _merge_tree(scratch[:], k, s // k, lanes)
    newl = jnp.minimum(length, k)
    orow = lax.broadcasted_iota(jnp.int32, (k, 1), 0)
    o_ref[0] = jnp.where(orow < newl, y, 0.0)


def _build(s, d_total, k, lanes, interpret=False):
    def call(x, lengths):
        bsz = x.shape[0]
        body = functools.partial(_topk_body, s=s, k=k, lanes=lanes)
        return pl.pallas_call(
            body,
            grid=(bsz, d_total // lanes),
            in_specs=[
                pl.BlockSpec(memory_space=pltpu.SMEM),
                pl.BlockSpec((1, s, lanes), lambda b, dt: (b, 0, dt)),
            ],
            out_specs=pl.BlockSpec((1, k, lanes), lambda b, dt: (b, 0, dt)),
            out_shape=jax.ShapeDtypeStruct((bsz, k, d_total), jnp.float32),
            scratch_shapes=[pltpu.VMEM((s, lanes), jnp.float32)],
            compiler_params=pltpu.CompilerParams(
                dimension_semantics=("parallel", "parallel"),
            ),
            interpret=interpret,
        )(lengths, x)

    return call


@jax.jit
def kernel(x, lengths):
    bsz, s, d_total = x.shape
    pooled = _build(s, d_total, _K, 128)(x, lengths)
    return pooled, jnp.minimum(lengths, _K)


# speedup vs baseline: 3.3943x; 3.3943x over previous
import functools

import jax
import jax.numpy as jnp
from jax import lax
from jax.experimental import pallas as pl
from jax.experimental.pallas import tpu as pltpu

_K = 512
_NEG = float("-inf")


def _stage(x, d, lanes, run=None, flip=None, asc=False):
    m = x.shape[0]
    if d >= 8:
        k = m // (2 * d)
        xr = x.reshape(k, 2, d, lanes)
        a = xr[:, 0]
        b = xr[:, 1]
        mx = jnp.maximum(a, b)
        mn = jnp.minimum(a, b)
        if run is None:
            lo, hi = (mn, mx) if asc else (mx, mn)
        else:
            run_shift = run.bit_length() - 1
            shift = run_shift - (2 * d).bit_length() + 1
            blk = lax.broadcasted_iota(jnp.int32, (k, 1, 1), 0)
            desc = ((blk >> shift) & 1) == 0
            if flip is not None:
                desc = desc != flip
            lo = jnp.where(desc, mx, mn)
            hi = jnp.where(desc, mn, mx)
        return jnp.concatenate([lo[:, None], hi[:, None]], axis=1).reshape(
            m, lanes
        )
    i = lax.broadcasted_iota(jnp.int32, (m, 1), 0)
    is_lower = (i & d) == 0
    partner = jnp.where(is_lower, jnp.roll(x, -d, axis=0), jnp.roll(x, d, axis=0))
    mx = jnp.maximum(x, partner)
    mn = jnp.minimum(x, partner)
    if run is None:
        want_max = (i & d) != 0 if asc else is_lower
    else:
        run_shift = run.bit_length() - 1
        desc = ((i >> run_shift) & 1) == 0
        if flip is not None:
            desc = desc != flip
        want_max = is_lower == desc
    return jnp.where(want_max, mx, mn)


def _sort_chunk(x, chunk, lanes, flip):
    run = 2
    while run <= chunk:
        d = run // 2
        while d >= 1:
            x = _stage(x, d, lanes, run=run, flip=flip)
            d //= 2
        run *= 2
    return x


def _merge_const(m, lanes, asc):
    nc, chunk, _ = m.shape
    x = m.reshape(nc * chunk, lanes)
    d = chunk // 2
    while d >= 1:
        x = _stage(x, d, lanes, asc=asc)
        d //= 2
    return x.reshape(nc, chunk, lanes)


def _merge_tree(x, chunk, nchunks, lanes):
    xr = x.reshape(nchunks // 2, 2, chunk, lanes)
    m = jnp.maximum(xr[:, 0], xr[:, 1])
    nm = nchunks // 2
    while nm > 1:
        mr = m.reshape(nm // 2, 2, chunk, lanes)
        dsc = _merge_const(mr[:, 0], lanes, asc=False)
        acs = _merge_const(mr[:, 1], lanes, asc=True)
        m = jnp.maximum(dsc, acs)
        nm //= 2
    return _merge_const(m, lanes, asc=False).reshape(chunk, lanes)


def _topk_body(len_ref, x_ref, o_ref, scratch, *, s, k, lanes):
    b = pl.program_id(0)
    length = len_ref[b]
    x = x_ref[0]
    row = lax.broadcasted_iota(jnp.int32, (s, 1), 0)
    scratch[:] = jnp.where(row < length, x, _NEG)

    nact = (length + (k - 1)) // k

    def chunk_body(c, carry):
        ch = scratch[pl.ds(c * k, k), :]
        scratch[pl.ds(c * k, k), :] = _sort_chunk(ch, k, lanes, (c & 1) == 1)
        return carry

    lax.fori_loop(0, nact, chunk_body, 0)
    y = _merge_tree(scratch[:], k, s // k, lanes)
    newl = jnp.minimum(length, k)
    orow = lax.broadcasted_iota(jnp.int32, (k, 1), 0)
    o_ref[0] = jnp.where(orow < newl, y, 0.0)


def _build(s, d_total, k, lanes, interpret=False):
    def call(x, lengths):
        bsz = x.shape[0]
        body = functools.partial(_topk_body, s=s, k=k, lanes=lanes)
        return pl.pallas_call(
            body,
            grid=(bsz, d_total // lanes),
            in_specs=[
                pl.BlockSpec(memory_space=pltpu.SMEM),
                pl.BlockSpec((1, s, lanes), lambda b, dt: (b, 0, dt)),
            ],
            out_specs=pl.BlockSpec((1, k, lanes), lambda b, dt: (b, 0, dt)),
            out_shape=jax.ShapeDtypeStruct((bsz, k, d_total), jnp.float32),
            scratch_shapes=[pltpu.VMEM((s, lanes), jnp.float32)],
            compiler_params=pltpu.CompilerParams(
                dimension_semantics=("parallel", "parallel"),
            ),
            interpret=interpret,
        )(lengths, x)

    return call


@jax.jit
def kernel(x, lengths):
    bsz, s, d_total = x.shape
    pooled = _build(s, d_total, _K, 128)(x, lengths)
    return pooled, jnp.minimum(lengths, _K)
